# baseline (speedup 1.0000x reference)
import functools

import jax
import jax.numpy as jnp
from jax.experimental import pallas as pl
from jax.experimental.pallas import tpu as pltpu

_PAD = 128


def _up_conv_kernel(x_ref, w_ref, b_ref, o_ref, xd_refs, *, H, W):
    Cin = x_ref.shape[1]
    W2 = 2 * W
    N2 = H * W2
    N4 = 2 * H * W2
    Cout = w_ref.shape[2]

    x2 = x_ref[0]
    zcol = jnp.zeros((Cin, 1), jnp.bfloat16)
    lane = jax.lax.broadcasted_iota(jnp.int32, (1, N2), 1)
    v = jnp.remainder(lane, W2)
    m_t = {
        -1: jnp.where(v != 0, jnp.concatenate([zcol, x2[:, :N2 - 1]], axis=1),
                      jnp.bfloat16(0)),
        0: x2,
        1: jnp.where(v != W2 - 1, jnp.concatenate([x2[:, 1:], zcol], axis=1),
                     jnp.bfloat16(0)),
    }

    zpad = jnp.zeros((Cin, _PAD), jnp.bfloat16)
    for t in (-1, 0, 1):
        xd = xd_refs[t + 1]
        xd[:, 0:_PAD] = zpad
        xd[:, _PAD + N4:] = zpad
        for p in range(H):
            chunk = m_t[t][:, W2 * p:W2 * (p + 1)]
            xd[:, _PAD + 2 * W2 * p:_PAD + 2 * W2 * (p + 1)] = (
                jnp.concatenate([chunk, chunk], axis=1))

    acc = jnp.zeros((Cout, N4), jnp.float32)
    for dy in range(3):
        for t in (-1, 0, 1):
            s = _PAD + W2 * (dy - 1)
            acc = acc + jnp.dot(w_ref[dy, t + 1],
                                xd_refs[t + 1][:, s:s + N4],
                                preferred_element_type=jnp.float32)

    o_ref[0] = acc + b_ref[:, 0:1]


def kernel(x_nchw, w_oihw, bias):
    B, Cin, H, W = x_nchw.shape
    Cout = w_oihw.shape[0]
    N2 = 2 * H * W
    N4 = 4 * H * W

    xcol = jnp.repeat(x_nchw.astype(jnp.bfloat16), 2, axis=3).reshape(B, Cin, N2)
    wk = jnp.transpose(w_oihw, (2, 3, 0, 1)).astype(jnp.bfloat16)
    bb = jnp.broadcast_to(bias.astype(jnp.float32)[:, None], (Cout, 128))

    flops = 2 * B * 9 * Cout * Cin * N4
    bytes_accessed = int(xcol.size * 2 + B * Cout * N4 * 4 + wk.size * 2)

    def _body(x_ref, w_ref, b_ref, o_ref, xd0, xd1, xd2):
        _up_conv_kernel(x_ref, w_ref, b_ref, o_ref, (xd0, xd1, xd2), H=H, W=W)

    out = pl.pallas_call(
        _body,
        out_shape=jax.ShapeDtypeStruct((B, Cout, N4), jnp.float32),
        grid=(B,),
        in_specs=[
            pl.BlockSpec((1, Cin, N2), lambda i: (i, 0, 0)),
            pl.BlockSpec((3, 3, Cout, Cin), lambda i: (0, 0, 0, 0)),
            pl.BlockSpec((Cout, 128), lambda i: (0, 0)),
        ],
        out_specs=pl.BlockSpec((1, Cout, N4), lambda i: (i, 0, 0)),
        scratch_shapes=[
            pltpu.VMEM((Cin, _PAD + N4 + _PAD), jnp.bfloat16),
            pltpu.VMEM((Cin, _PAD + N4 + _PAD), jnp.bfloat16),
            pltpu.VMEM((Cin, _PAD + N4 + _PAD), jnp.bfloat16),
        ],
        compiler_params=pltpu.CompilerParams(
            dimension_semantics=("parallel",),
            vmem_limit_bytes=56 * 1024 * 1024),
        cost_estimate=pl.CostEstimate(
            flops=flops, transcendentals=0, bytes_accessed=bytes_accessed),
    )(xcol, wk, bb)

    return out.reshape(B, Cout, 2 * H, 2 * W)


# speedup vs baseline: 1.3236x; 1.3236x over previous
import functools

import jax
import jax.numpy as jnp
from jax.experimental import pallas as pl
from jax.experimental.pallas import tpu as pltpu

_PAD = 128


def _up_conv_kernel(x_ref, d_ref, w_ref, b_ref, o_ref, se_ref, so_ref, *, H, W):
    Cin = x_ref.shape[1]
    W2 = 2 * W
    N2 = H * W2
    N4 = 2 * N2
    Cout = w_ref.shape[1]

    xb = x_ref[0].astype(jnp.bfloat16)
    xcol = jnp.dot(xb, d_ref[...],
                   preferred_element_type=jnp.float32).astype(jnp.bfloat16)

    z1 = jnp.zeros((Cin, 1), jnp.bfloat16)
    v = jnp.remainder(jax.lax.broadcasted_iota(jnp.int32, (1, N2), 1), W2)
    m = {
        0: jnp.where(v != 0, jnp.concatenate([z1, xcol[:, :N2 - 1]], axis=1),
                     jnp.bfloat16(0)),
        1: xcol,
        2: jnp.where(v != W2 - 1, jnp.concatenate([xcol[:, 1:], z1], axis=1),
                     jnp.bfloat16(0)),
    }

    z64 = jnp.zeros((Cin, W2), jnp.bfloat16)
    for dx in range(3):
        r0 = dx * Cin
        chunks = [m[dx][:, W2 * p:W2 * (p + 1)] for p in range(H)]
        so_ref[r0:r0 + Cin, 0:W2] = z64
        so_ref[r0:r0 + Cin, W2:2 * W2] = chunks[0]
        for p in range(H):
            nxt = chunks[p + 1] if p + 1 < H else z64
            se_ref[r0:r0 + Cin, _PAD + 2 * W2 * p:_PAD + 2 * W2 * (p + 1)] = (
                jnp.concatenate([chunks[p], chunks[p]], axis=1))
            so_ref[r0:r0 + Cin, _PAD + 2 * W2 * p:_PAD + 2 * W2 * (p + 1)] = (
                jnp.concatenate([chunks[p], nxt], axis=1))

    acc = jnp.dot(w_ref[1], se_ref[:, _PAD:_PAD + N4],
                  preferred_element_type=jnp.float32)
    acc = acc + jnp.dot(w_ref[2], so_ref[:, _PAD:_PAD + N4],
                        preferred_element_type=jnp.float32)
    acc = acc + jnp.dot(w_ref[0], so_ref[:, 0:N4],
                        preferred_element_type=jnp.float32)

    o_ref[0] = acc + b_ref[:, 0:1]


def _dilation_matrix(H, W):
    d0 = jnp.repeat(jnp.eye(W, dtype=jnp.float32), 2, axis=1)
    return jnp.kron(jnp.eye(H, dtype=jnp.float32), d0)


def kernel(x_nchw, w_oihw, bias):
    B, Cin, H, W = x_nchw.shape
    Cout = w_oihw.shape[0]
    N = H * W

    x3 = x_nchw.reshape(B, Cin, N)
    dd = _dilation_matrix(H, W).astype(jnp.bfloat16)
    wk = jnp.transpose(w_oihw, (2, 0, 3, 1)).reshape(3, Cout, 3 * Cin)
    wk = wk.astype(jnp.bfloat16)
    bb = jnp.broadcast_to(bias.astype(jnp.float32)[:, None], (Cout, 128))

    flops = 2 * B * (N * 2 * N + 3 * 3 * Cin * Cout * 4 * N)
    bytes_accessed = int(x3.size * 4 + B * Cout * 4 * N * 4 + wk.size * 2)

    out = pl.pallas_call(
        functools.partial(_up_conv_kernel, H=H, W=W),
        out_shape=jax.ShapeDtypeStruct((B, Cout, 4 * N), jnp.float32),
        grid=(B,),
        in_specs=[
            pl.BlockSpec((1, Cin, N), lambda i: (i, 0, 0)),
            pl.BlockSpec((N, 2 * N), lambda i: (0, 0)),
            pl.BlockSpec((3, Cout, 3 * Cin), lambda i: (0, 0, 0)),
            pl.BlockSpec((Cout, 128), lambda i: (0, 0)),
        ],
        out_specs=pl.BlockSpec((1, Cout, 4 * N), lambda i: (i, 0, 0)),
        scratch_shapes=[
            pltpu.VMEM((3 * Cin, _PAD + 4 * N + _PAD), jnp.bfloat16),
            pltpu.VMEM((3 * Cin, _PAD + 4 * N + _PAD), jnp.bfloat16),
        ],
        compiler_params=pltpu.CompilerParams(
            dimension_semantics=("parallel",),
            vmem_limit_bytes=56 * 1024 * 1024),
        cost_estimate=pl.CostEstimate(
            flops=flops, transcendentals=0, bytes_accessed=bytes_accessed),
    )(x3, dd, wk, bb)

    return out.reshape(B, Cout, 2 * H, 2 * W)
